# baseline (speedup 1.0000x reference)
import functools

import jax
import jax.numpy as jnp
from jax import lax
from jax.experimental import pallas as pl
from jax.experimental.pallas import tpu as pltpu
from jax.experimental.pallas import tpu_sc as plsc

_EMBED_DIM = 16
_NUM_CORES = 2
_NUM_SUBCORES = 16
_NUM_WORKERS = _NUM_CORES * _NUM_SUBCORES


@functools.partial(jax.jit, static_argnums=(2,))
def _gather_rows(flat_idx, table, n):
    n_per_w = n // _NUM_WORKERS
    mesh = plsc.VectorSubcoreMesh(core_axis_name="c", subcore_axis_name="s")

    @functools.partial(
        pl.kernel,
        mesh=mesh,
        compiler_params=pltpu.CompilerParams(use_tc_tiling_on_sc=False),
        out_type=jax.ShapeDtypeStruct((n, _EMBED_DIM), jnp.float32),
        scratch_types=[
            pltpu.VMEM((n_per_w,), jnp.int32),
            pltpu.VMEM((n_per_w, _EMBED_DIM), jnp.float32),
            pltpu.SemaphoreType.DMA,
        ],
    )
    def gather_kernel(idx_hbm, table_hbm, out_hbm, idx_v, rows_v, sem):
        wid = lax.axis_index("s") * _NUM_CORES + lax.axis_index("c")
        base = wid * n_per_w
        pltpu.sync_copy(idx_hbm.at[pl.ds(base, n_per_w)], idx_v)
        pltpu.async_copy(table_hbm.at[idx_v], rows_v, sem).wait()
        pltpu.sync_copy(rows_v, out_hbm.at[pl.ds(base, n_per_w)])

    return gather_kernel(flat_idx, table)


def kernel(inputs, table):
    batch, cars = inputs.shape
    n = batch * cars
    rows = _gather_rows(inputs.reshape(n), table, n)
    return rows.reshape(batch, 1, cars * _EMBED_DIM)


# speedup vs baseline: 1.2428x; 1.2428x over previous
import functools

import jax
import jax.numpy as jnp
from jax import lax
from jax.experimental import pallas as pl
from jax.experimental.pallas import tpu as pltpu
from jax.experimental.pallas import tpu_sc as plsc

_D = 16
_NC = 2
_NS = 16
_NW = _NC * _NS


@jax.jit
def _split_layer(inputs, table):
    batch, cars = inputs.shape
    rows_per_w = batch // _NW
    out_w = cars * _D
    mesh = plsc.VectorSubcoreMesh(core_axis_name="c", subcore_axis_name="s")

    @functools.partial(
        pl.kernel,
        mesh=mesh,
        out_type=jax.ShapeDtypeStruct((batch, 1, out_w), jnp.float32),
        scratch_types=[
            pltpu.VMEM((rows_per_w, cars), jnp.int32),
            pltpu.VMEM((1, out_w), jnp.float32),
            pltpu.SemaphoreType.DMA,
        ],
    )
    def sc_kernel(idx_hbm, table_hbm, out_hbm, idx_v, rowbuf, gsem):
        wid = lax.axis_index("s") * _NC + lax.axis_index("c")
        row0 = wid * rows_per_w
        pltpu.sync_copy(idx_hbm.at[pl.ds(row0, rows_per_w)], idx_v)

        def body(r, carry):
            va = idx_v[r, pl.ds(0, _D)]
            vb = idx_v[r, pl.ds(cars - _D, _D)]
            copies = []
            for j in range(cars):
                idx = va[j] if j < _D else vb[j - (cars - _D)]
                copies.append(pltpu.async_copy(
                    table_hbm.at[idx], rowbuf.at[0, pl.ds(j * _D, _D)], gsem))
            for c in copies:
                c.wait()
            pltpu.sync_copy(rowbuf.at[0], out_hbm.at[row0 + r, 0])
            return carry

        lax.fori_loop(0, rows_per_w, body, 0)

    return sc_kernel(inputs, table)


def kernel(inputs, table):
    return _split_layer(inputs, table)


# speedup vs baseline: 1.4044x; 1.1301x over previous
import functools

import jax
import jax.numpy as jnp
from jax import lax
from jax.experimental import pallas as pl
from jax.experimental.pallas import tpu as pltpu
from jax.experimental.pallas import tpu_sc as plsc

_D = 16
_NC = 2
_NS = 16
_NW = _NC * _NS
_G = 4
_NG = 2


@jax.jit
def _split_layer(inputs, table):
    batch, cars = inputs.shape
    rows_per_w = batch // _NW
    out_w = cars * _D
    step = _G * _NG
    mesh = plsc.VectorSubcoreMesh(core_axis_name="c", subcore_axis_name="s")

    @functools.partial(
        pl.kernel,
        mesh=mesh,
        out_type=jax.ShapeDtypeStruct((batch, 1, out_w), jnp.float32),
        scratch_types=[
            pltpu.VMEM((rows_per_w, cars), jnp.int32),
            pltpu.VMEM((step, out_w), jnp.float32),
        ]
        + [pltpu.SemaphoreType.DMA for _ in range(2 * _NG)],
    )
    def sc_kernel(idx_hbm, table_hbm, out_hbm, idx_v, rowbuf, *sems):
        gsem = sems[:_NG]
        osem = sems[_NG:]
        wid = lax.axis_index("s") * _NC + lax.axis_index("c")
        row0 = wid * rows_per_w
        pltpu.sync_copy(idx_hbm.at[pl.ds(row0, rows_per_w)], idx_v)

        def body(q, carry):
            base = q * step

            def fire(g):
                copies = []
                for t in range(_G):
                    r = base + g * _G + t
                    va = idx_v[r, pl.ds(0, _D)]
                    vb = idx_v[r, pl.ds(cars - _D, _D)]
                    for j in range(cars):
                        idx = va[j] if j < _D else vb[j - (cars - _D)]
                        copies.append(pltpu.async_copy(
                            table_hbm.at[idx],
                            rowbuf.at[g * _G + t, pl.ds(j * _D, _D)],
                            gsem[g]))
                return copies

            def drain(g, copies):
                outs = []
                for c in copies:
                    c.wait()
                for t in range(_G):
                    r = base + g * _G + t
                    outs.append(pltpu.async_copy(
                        rowbuf.at[g * _G + t], out_hbm.at[row0 + r, 0],
                        osem[g]))
                return outs

            fired = [fire(g) for g in range(_NG)]
            written = [drain(g, fired[g]) for g in range(_NG)]
            for outs in written:
                for o in outs:
                    o.wait()
            return carry

        lax.fori_loop(0, rows_per_w // step, body, 0)

    return sc_kernel(inputs, table)


def kernel(inputs, table):
    return _split_layer(inputs, table)
